# baseline (speedup 1.0000x reference)
import functools

import jax
import jax.numpy as jnp
from jax import lax
from jax.experimental import pallas as pl
from jax.experimental.pallas import tpu as pltpu

BLK = 1024


def _dense1_body(tab_ref, pa_ref, pb_ref, ws1t_ref, wn1t_ref, b1_ref, g1_ref,
                 bt1_ref, wn2t_ref, n_ref, h_ref, p0_ref, p1_ref):
    ps = pa_ref[...] + pb_ref[...]
    cnt = ps[:, 12:13]
    inv = 1.0 / jnp.maximum(cnt, 1.0)
    agg = ps[:, :12] * inv
    xb = tab_ref[:, :12]
    h = (jnp.dot(xb, ws1t_ref[...], preferred_element_type=jnp.float32)
         + jnp.dot(agg, wn1t_ref[...], preferred_element_type=jnp.float32)
         + b1_ref[...])
    mu = jnp.mean(h, axis=-1, keepdims=True)
    var = jnp.mean((h - mu) ** 2, axis=-1, keepdims=True)
    h = (h - mu) * lax.rsqrt(var + 1e-5) * g1_ref[...] + bt1_ref[...]
    h = jnp.maximum(h, 0.0)
    row0 = pl.program_id(0) * BLK
    rows = row0 + lax.broadcasted_iota(jnp.int32, (BLK, 1), 0)
    h = jnp.where(rows < n_ref[0], h, 0.0)
    h_ref[...] = h
    p = jnp.dot(h, wn2t_ref[...], preferred_element_type=jnp.float32)
    p0_ref[...] = p[:, :16]
    p1_ref[...] = p[:, 16:]


def _dense2_body(h_ref, pa_ref, pb_ref, q0_ref, q1_ref, ws2t_ref, b2_ref,
                 g2_ref, bt2_ref, out_ref):
    cnt = pa_ref[:, 12:13] + pb_ref[:, 12:13]
    inv = 1.0 / jnp.maximum(cnt, 1.0)
    agg = jnp.concatenate([q0_ref[...], q1_ref[...]], axis=1) * inv
    h = (jnp.dot(h_ref[...], ws2t_ref[...], preferred_element_type=jnp.float32)
         + agg + b2_ref[...])
    mu = jnp.mean(h, axis=-1, keepdims=True)
    var = jnp.mean((h - mu) ** 2, axis=-1, keepdims=True)
    h = (h - mu) * lax.rsqrt(var + 1e-5) * g2_ref[...] + bt2_ref[...]
    out_ref[...] = jnp.maximum(h, 0.0)


def _blk_spec(cols):
    return pl.BlockSpec((BLK, cols), lambda i: (i, 0))


def _full_spec(shape):
    return pl.BlockSpec(shape, lambda i: tuple(0 for _ in shape))


def _dense1(np_, tab1, part_a, part_b, ws1t, wn1t, b1, g1, bt1, wn2t, n_nodes):
    grid = (np_ // BLK,)
    return pl.pallas_call(
        _dense1_body,
        grid=grid,
        in_specs=[
            _blk_spec(16), _blk_spec(16), _blk_spec(16),
            _full_spec((12, 64)), _full_spec((12, 64)), _full_spec((1, 64)),
            _full_spec((1, 64)), _full_spec((1, 64)), _full_spec((64, 32)),
            pl.BlockSpec(memory_space=pltpu.SMEM),
        ],
        out_specs=[_blk_spec(64), _blk_spec(16), _blk_spec(16)],
        out_shape=[
            jax.ShapeDtypeStruct((np_, 64), jnp.float32),
            jax.ShapeDtypeStruct((np_, 16), jnp.float32),
            jax.ShapeDtypeStruct((np_, 16), jnp.float32),
        ],
    )(tab1, part_a, part_b, ws1t, wn1t, b1, g1, bt1, wn2t,
      jnp.full((1,), n_nodes, jnp.int32))


def _dense2(np_, h, part_a, part_b, q0, q1, ws2t, b2, g2, bt2):
    grid = (np_ // BLK,)
    return pl.pallas_call(
        _dense2_body,
        grid=grid,
        in_specs=[
            _blk_spec(64), _blk_spec(16), _blk_spec(16), _blk_spec(16),
            _blk_spec(16),
            _full_spec((64, 32)), _full_spec((1, 32)), _full_spec((1, 32)),
            _full_spec((1, 32)),
        ],
        out_specs=_blk_spec(32),
        out_shape=jax.ShapeDtypeStruct((np_, 32), jnp.float32),
    )(h, part_a, part_b, q0, q1, ws2t, b2, g2, bt2)


def kernel(x, edge_index, W_self1, W_neigh1, b1, g1, beta1,
           W_self2, W_neigh2, b2, g2, beta2):
    n = x.shape[0]
    np_ = ((n + 1 + BLK - 1) // BLK) * BLK
    s = edge_index[0]
    d = edge_index[1]

    ones = jnp.ones((n, 1), jnp.float32)
    zeros3 = jnp.zeros((n, 3), jnp.float32)
    tab1 = jnp.concatenate([x, ones, zeros3], axis=1)
    tab1 = jnp.concatenate([tab1, jnp.zeros((np_ - n, 16), jnp.float32)], axis=0)

    part_a = jax.ops.segment_sum(tab1[s], d, num_segments=np_)
    part_b = jnp.zeros_like(part_a)

    h, p0, p1 = _dense1(np_, tab1, part_a, part_b,
                        W_self1.T, W_neigh1.T, b1.reshape(1, 64),
                        g1.reshape(1, 64), beta1.reshape(1, 64),
                        W_neigh2.T, n)

    q0 = jax.ops.segment_sum(p0[s], d, num_segments=np_)
    q1 = jax.ops.segment_sum(p1[s], d, num_segments=np_)

    out = _dense2(np_, h, part_a, part_b, q0, q1,
                  W_self2.T, b2.reshape(1, 32), g2.reshape(1, 32),
                  beta2.reshape(1, 32))
    return out[:n]


# speedup vs baseline: 30.8072x; 30.8072x over previous
import functools

import jax
import jax.numpy as jnp
from jax import lax
from jax.experimental import pallas as pl
from jax.experimental.pallas import tpu as pltpu
from jax.experimental.pallas import tpu_sc as plsc

BLK = 1024
NC = 2
NT = 16
U = 8


def _make_agg(two_tables, np_, n_rows):
    mesh = plsc.VectorSubcoreMesh(core_axis_name="c", subcore_axis_name="s")
    node_rows = np_ // NT
    stg = 512
    nfull, rem = divmod(node_rows, stg)
    if two_tables:
        tile_rows = n_rows // NT
    else:
        tile_rows = n_rows // (NC * NT)
    steps = tile_rows // U

    def body(tab_hbm, s_hbm, d_hbm, out_hbm, s_v, d_v, rows_v, stage_v,
             acc_sh, sem):
        c = lax.axis_index("c")
        t = lax.axis_index("s")

        def zrow(i, carry):
            stage_v[i, :] = jnp.zeros((16,), jnp.float32)
            return carry
        lax.fori_loop(0, stg, zrow, 0)
        nbase = t * node_rows
        for k in range(nfull):
            pltpu.sync_copy(stage_v, acc_sh.at[pl.ds(nbase + k * stg, stg)])
        if rem:
            pltpu.sync_copy(stage_v.at[pl.ds(0, rem)],
                            acc_sh.at[pl.ds(nbase + nfull * stg, rem)])
        plsc.subcore_barrier()

        if two_tables:
            d_row0 = t * tile_rows
            s_row0 = c * n_rows + d_row0
        else:
            s_row0 = (c * NT + t) * tile_rows
            d_row0 = s_row0

        def step(g, carry):
            pltpu.sync_copy(s_hbm.at[pl.ds(s_row0 + g * U, U)], s_v)
            pltpu.sync_copy(d_hbm.at[pl.ds(d_row0 + g * U, U)], d_v)
            descs = [pltpu.async_copy(tab_hbm.at[s_v.at[j]], rows_v.at[j], sem)
                     for j in range(U)]
            for dsc in descs:
                dsc.wait()
            for j in range(U):
                pltpu.sync_copy(rows_v.at[j], acc_sh.at[d_v.at[j]], add=True)
            return carry
        lax.fori_loop(0, steps, step, 0)
        plsc.subcore_barrier()

        obase = c * np_ + nbase
        for k in range(nfull):
            pltpu.sync_copy(acc_sh.at[pl.ds(nbase + k * stg, stg)], stage_v)
            pltpu.sync_copy(stage_v, out_hbm.at[pl.ds(obase + k * stg, stg)])
        if rem:
            pltpu.sync_copy(acc_sh.at[pl.ds(nbase + nfull * stg, rem)],
                            stage_v.at[pl.ds(0, rem)])
            pltpu.sync_copy(stage_v.at[pl.ds(0, rem)],
                            out_hbm.at[pl.ds(obase + nfull * stg, rem)])

    return pl.kernel(
        body,
        out_type=jax.ShapeDtypeStruct((NC * np_, 16), jnp.float32),
        mesh=mesh,
        scratch_types=[
            pltpu.VMEM((U, 128), jnp.int32),
            pltpu.VMEM((U, 128), jnp.int32),
            pltpu.VMEM((U, 128, 16), jnp.float32),
            pltpu.VMEM((512, 16), jnp.float32),
            pltpu.VMEM_SHARED((np_, 16), jnp.float32),
            pltpu.SemaphoreType.DMA,
        ],
        compiler_params=pltpu.CompilerParams(use_tc_tiling_on_sc=False),
    )


def _dense1_body(tab_ref, pa_ref, pb_ref, ws1t_ref, wn1t_ref, b1_ref, g1_ref,
                 bt1_ref, wn2t_ref, n_ref, h_ref, p_ref):
    ps = pa_ref[...] + pb_ref[...]
    cnt = ps[:, 12:13]
    inv = 1.0 / jnp.maximum(cnt, 1.0)
    agg = ps[:, :12] * inv
    xb = tab_ref[:, :12]
    h = (jnp.dot(xb, ws1t_ref[...], preferred_element_type=jnp.float32)
         + jnp.dot(agg, wn1t_ref[...], preferred_element_type=jnp.float32)
         + b1_ref[...])
    mu = jnp.mean(h, axis=-1, keepdims=True)
    var = jnp.mean((h - mu) ** 2, axis=-1, keepdims=True)
    h = (h - mu) * lax.rsqrt(var + 1e-5) * g1_ref[...] + bt1_ref[...]
    h = jnp.maximum(h, 0.0)
    row0 = pl.program_id(0) * BLK
    rows = row0 + lax.broadcasted_iota(jnp.int32, (BLK, 1), 0)
    h = jnp.where(rows < n_ref[0], h, 0.0)
    h_ref[...] = h
    p = jnp.dot(h, wn2t_ref[...], preferred_element_type=jnp.float32)
    p_ref[0] = p[:, :16]
    p_ref[1] = p[:, 16:]


def _dense2_body(h_ref, pa_ref, pb_ref, q0_ref, q1_ref, ws2t_ref, b2_ref,
                 g2_ref, bt2_ref, out_ref):
    cnt = pa_ref[:, 12:13] + pb_ref[:, 12:13]
    inv = 1.0 / jnp.maximum(cnt, 1.0)
    agg = jnp.concatenate([q0_ref[...], q1_ref[...]], axis=1) * inv
    h = (jnp.dot(h_ref[...], ws2t_ref[...], preferred_element_type=jnp.float32)
         + agg + b2_ref[...])
    mu = jnp.mean(h, axis=-1, keepdims=True)
    var = jnp.mean((h - mu) ** 2, axis=-1, keepdims=True)
    h = (h - mu) * lax.rsqrt(var + 1e-5) * g2_ref[...] + bt2_ref[...]
    out_ref[...] = jnp.maximum(h, 0.0)


def _blk_spec(cols, row_off=0):
    return pl.BlockSpec((BLK, cols), lambda i, _o=row_off: (i + _o, 0))


def _full_spec(shape):
    return pl.BlockSpec(shape, lambda i: tuple(0 for _ in shape))


def _dense1(np_, tab1, parts, ws1t, wn1t, b1, g1, bt1, wn2t, n_nodes):
    nb = np_ // BLK
    return pl.pallas_call(
        _dense1_body,
        grid=(nb,),
        in_specs=[
            _blk_spec(16), _blk_spec(16), _blk_spec(16, nb),
            _full_spec((12, 64)), _full_spec((12, 64)), _full_spec((1, 64)),
            _full_spec((1, 64)), _full_spec((1, 64)), _full_spec((64, 32)),
            pl.BlockSpec(memory_space=pltpu.SMEM),
        ],
        out_specs=[
            _blk_spec(64),
            pl.BlockSpec((2, BLK, 16), lambda i: (0, i, 0)),
        ],
        out_shape=[
            jax.ShapeDtypeStruct((np_, 64), jnp.float32),
            jax.ShapeDtypeStruct((2, np_, 16), jnp.float32),
        ],
    )(tab1, parts, parts, ws1t, wn1t, b1, g1, bt1, wn2t,
      jnp.full((1,), n_nodes, jnp.int32))


def _dense2(np_, h, parts, q, ws2t, b2, g2, bt2):
    nb = np_ // BLK
    return pl.pallas_call(
        _dense2_body,
        grid=(nb,),
        in_specs=[
            _blk_spec(64), _blk_spec(16), _blk_spec(16, nb),
            _blk_spec(16), _blk_spec(16, nb),
            _full_spec((64, 32)), _full_spec((1, 32)), _full_spec((1, 32)),
            _full_spec((1, 32)),
        ],
        out_specs=_blk_spec(32),
        out_shape=jax.ShapeDtypeStruct((np_, 32), jnp.float32),
    )(h, parts, parts, q, q, ws2t, b2, g2, bt2)


def kernel(x, edge_index, W_self1, W_neigh1, b1, g1, beta1,
           W_self2, W_neigh2, b2, g2, beta2):
    n = x.shape[0]
    np_ = -(-(n + 1) // BLK) * BLK
    s = edge_index[0]
    d = edge_index[1]
    e = s.shape[0]
    gran = NC * NT * 128 * U
    e_pad = -(-e // gran) * gran
    r = e_pad // 128

    pad = jnp.full((e_pad - e,), n, jnp.int32)
    s2d = jnp.concatenate([s, pad]).reshape(r, 128)
    d2d = jnp.concatenate([d, pad]).reshape(r, 128)

    tab1 = jnp.concatenate(
        [x, jnp.ones((n, 1), jnp.float32), jnp.zeros((n, 3), jnp.float32)],
        axis=1)
    tab1 = jnp.concatenate(
        [tab1, jnp.zeros((np_ - n, 16), jnp.float32)], axis=0)

    parts = _make_agg(False, np_, r)(tab1, s2d, d2d)

    h, p = _dense1(np_, tab1, parts,
                   W_self1.T, W_neigh1.T, b1.reshape(1, 64),
                   g1.reshape(1, 64), beta1.reshape(1, 64),
                   W_neigh2.T, n)

    tab2 = p.reshape(2 * np_, 16)
    s_off = jnp.concatenate([s2d, s2d + np_], axis=0)
    q = _make_agg(True, np_, r)(tab2, s_off, d2d)

    out = _dense2(np_, h, parts, q,
                  W_self2.T, b2.reshape(1, 32), g2.reshape(1, 32),
                  beta2.reshape(1, 32))
    return out[:n]


# speedup vs baseline: 39.0772x; 1.2684x over previous
import functools

import jax
import jax.numpy as jnp
from jax import lax
from jax.experimental import pallas as pl
from jax.experimental.pallas import tpu as pltpu
from jax.experimental.pallas import tpu_sc as plsc

BLK = 1024
NC = 2
NT = 16
U = 4


def _make_agg(two_tables, np_, n_rows):
    mesh = plsc.VectorSubcoreMesh(core_axis_name="c", subcore_axis_name="s")
    node_rows = np_ // NT
    stg = 256
    nfull, rem = divmod(node_rows, stg)
    if two_tables:
        tile_rows = n_rows // NT
    else:
        tile_rows = n_rows // (NC * NT)
    steps = tile_rows // U
    pairs = steps // 2

    def body(tab_hbm, s_hbm, d_hbm, out_hbm, s_v, d_v, rows_v, stage_v,
             acc_sh, sem, sem2):
        c = lax.axis_index("c")
        t = lax.axis_index("s")

        def zrow(i, carry):
            stage_v[i, :] = jnp.zeros((16,), jnp.float32)
            return carry
        lax.fori_loop(0, stg, zrow, 0)
        nbase = t * node_rows
        for k in range(nfull):
            pltpu.sync_copy(stage_v, acc_sh.at[pl.ds(nbase + k * stg, stg)])
        if rem:
            pltpu.sync_copy(stage_v.at[pl.ds(0, rem)],
                            acc_sh.at[pl.ds(nbase + nfull * stg, rem)])
        plsc.subcore_barrier()

        if two_tables:
            d_row0 = t * tile_rows
            s_row0 = c * n_rows + d_row0
        else:
            s_row0 = (c * NT + t) * tile_rows
            d_row0 = s_row0

        def fire(g, buf):
            pltpu.sync_copy(s_hbm.at[pl.ds(s_row0 + g * U, U)], s_v.at[buf])
            pltpu.sync_copy(d_hbm.at[pl.ds(d_row0 + g * U, U)], d_v.at[buf])
            for j in range(U):
                pltpu.async_copy(tab_hbm.at[s_v.at[buf, j]],
                                 rows_v.at[buf, j], sem)

        def drain_scatter(buf):
            for j in range(U):
                pltpu.make_async_copy(tab_hbm.at[s_v.at[buf, j]],
                                      rows_v.at[buf, j], sem).wait()
            descs = [pltpu.async_copy(rows_v.at[buf, j],
                                      acc_sh.at[d_v.at[buf, j]], sem2,
                                      add=True)
                     for j in range(U)]
            for dsc in descs:
                dsc.wait()

        fire(0, 0)

        def pair(gg, carry):
            g0 = 2 * gg
            fire(g0 + 1, 1)
            drain_scatter(0)

            @pl.when(gg + 1 < pairs)
            def _():
                fire(g0 + 2, 0)

            drain_scatter(1)
            return carry
        lax.fori_loop(0, pairs, pair, 0)
        plsc.subcore_barrier()

        obase = c * np_ + nbase
        for k in range(nfull):
            pltpu.sync_copy(acc_sh.at[pl.ds(nbase + k * stg, stg)], stage_v)
            pltpu.sync_copy(stage_v, out_hbm.at[pl.ds(obase + k * stg, stg)])
        if rem:
            pltpu.sync_copy(acc_sh.at[pl.ds(nbase + nfull * stg, rem)],
                            stage_v.at[pl.ds(0, rem)])
            pltpu.sync_copy(stage_v.at[pl.ds(0, rem)],
                            out_hbm.at[pl.ds(obase + nfull * stg, rem)])

    return pl.kernel(
        body,
        out_type=jax.ShapeDtypeStruct((NC * np_, 16), jnp.float32),
        mesh=mesh,
        scratch_types=[
            pltpu.VMEM((2, U, 128), jnp.int32),
            pltpu.VMEM((2, U, 128), jnp.int32),
            pltpu.VMEM((2, U, 128, 16), jnp.float32),
            pltpu.VMEM((256, 16), jnp.float32),
            pltpu.VMEM_SHARED((np_, 16), jnp.float32),
            pltpu.SemaphoreType.DMA,
            pltpu.SemaphoreType.DMA,
        ],
        compiler_params=pltpu.CompilerParams(use_tc_tiling_on_sc=False),
    )


def _dense1_body(tab_ref, pa_ref, pb_ref, ws1t_ref, wn1t_ref, b1_ref, g1_ref,
                 bt1_ref, wn2t_ref, n_ref, h_ref, p_ref):
    ps = pa_ref[...] + pb_ref[...]
    cnt = ps[:, 12:13]
    inv = 1.0 / jnp.maximum(cnt, 1.0)
    agg = ps[:, :12] * inv
    xb = tab_ref[:, :12]
    h = (jnp.dot(xb, ws1t_ref[...], preferred_element_type=jnp.float32)
         + jnp.dot(agg, wn1t_ref[...], preferred_element_type=jnp.float32)
         + b1_ref[...])
    mu = jnp.mean(h, axis=-1, keepdims=True)
    var = jnp.mean((h - mu) ** 2, axis=-1, keepdims=True)
    h = (h - mu) * lax.rsqrt(var + 1e-5) * g1_ref[...] + bt1_ref[...]
    h = jnp.maximum(h, 0.0)
    row0 = pl.program_id(0) * BLK
    rows = row0 + lax.broadcasted_iota(jnp.int32, (BLK, 1), 0)
    h = jnp.where(rows < n_ref[0], h, 0.0)
    h_ref[...] = h
    p = jnp.dot(h, wn2t_ref[...], preferred_element_type=jnp.float32)
    p_ref[0] = p[:, :16]
    p_ref[1] = p[:, 16:]


def _dense2_body(h_ref, pa_ref, pb_ref, q0_ref, q1_ref, ws2t_ref, b2_ref,
                 g2_ref, bt2_ref, out_ref):
    cnt = pa_ref[:, 12:13] + pb_ref[:, 12:13]
    inv = 1.0 / jnp.maximum(cnt, 1.0)
    agg = jnp.concatenate([q0_ref[...], q1_ref[...]], axis=1) * inv
    h = (jnp.dot(h_ref[...], ws2t_ref[...], preferred_element_type=jnp.float32)
         + agg + b2_ref[...])
    mu = jnp.mean(h, axis=-1, keepdims=True)
    var = jnp.mean((h - mu) ** 2, axis=-1, keepdims=True)
    h = (h - mu) * lax.rsqrt(var + 1e-5) * g2_ref[...] + bt2_ref[...]
    out_ref[...] = jnp.maximum(h, 0.0)


def _blk_spec(cols, row_off=0):
    return pl.BlockSpec((BLK, cols), lambda i, _o=row_off: (i + _o, 0))


def _full_spec(shape):
    return pl.BlockSpec(shape, lambda i: tuple(0 for _ in shape))


def _dense1(np_, tab1, parts, ws1t, wn1t, b1, g1, bt1, wn2t, n_nodes):
    nb = np_ // BLK
    return pl.pallas_call(
        _dense1_body,
        grid=(nb,),
        in_specs=[
            _blk_spec(16), _blk_spec(16), _blk_spec(16, nb),
            _full_spec((12, 64)), _full_spec((12, 64)), _full_spec((1, 64)),
            _full_spec((1, 64)), _full_spec((1, 64)), _full_spec((64, 32)),
            pl.BlockSpec(memory_space=pltpu.SMEM),
        ],
        out_specs=[
            _blk_spec(64),
            pl.BlockSpec((2, BLK, 16), lambda i: (0, i, 0)),
        ],
        out_shape=[
            jax.ShapeDtypeStruct((np_, 64), jnp.float32),
            jax.ShapeDtypeStruct((2, np_, 16), jnp.float32),
        ],
    )(tab1, parts, parts, ws1t, wn1t, b1, g1, bt1, wn2t,
      jnp.full((1,), n_nodes, jnp.int32))


def _dense2(np_, h, parts, q, ws2t, b2, g2, bt2):
    nb = np_ // BLK
    return pl.pallas_call(
        _dense2_body,
        grid=(nb,),
        in_specs=[
            _blk_spec(64), _blk_spec(16), _blk_spec(16, nb),
            _blk_spec(16), _blk_spec(16, nb),
            _full_spec((64, 32)), _full_spec((1, 32)), _full_spec((1, 32)),
            _full_spec((1, 32)),
        ],
        out_specs=_blk_spec(32),
        out_shape=jax.ShapeDtypeStruct((np_, 32), jnp.float32),
    )(h, parts, parts, q, q, ws2t, b2, g2, bt2)


def kernel(x, edge_index, W_self1, W_neigh1, b1, g1, beta1,
           W_self2, W_neigh2, b2, g2, beta2):
    n = x.shape[0]
    np_ = -(-(n + 1) // BLK) * BLK
    s = edge_index[0]
    d = edge_index[1]
    e = s.shape[0]
    gran = NC * NT * 128 * U * 2
    e_pad = -(-e // gran) * gran
    r = e_pad // 128

    pad = jnp.full((e_pad - e,), n, jnp.int32)
    s2d = jnp.concatenate([s, pad]).reshape(r, 128)
    d2d = jnp.concatenate([d, pad]).reshape(r, 128)

    tab1 = jnp.concatenate(
        [x, jnp.ones((n, 1), jnp.float32), jnp.zeros((n, 3), jnp.float32)],
        axis=1)
    tab1 = jnp.concatenate(
        [tab1, jnp.zeros((np_ - n, 16), jnp.float32)], axis=0)

    parts = _make_agg(False, np_, r)(tab1, s2d, d2d)

    h, p = _dense1(np_, tab1, parts,
                   W_self1.T, W_neigh1.T, b1.reshape(1, 64),
                   g1.reshape(1, 64), beta1.reshape(1, 64),
                   W_neigh2.T, n)

    tab2 = p.reshape(2 * np_, 16)
    s_off = jnp.concatenate([s2d, s2d + np_], axis=0)
    q = _make_agg(True, np_, r)(tab2, s_off, d2d)

    out = _dense2(np_, h, parts, q,
                  W_self2.T, b2.reshape(1, 32), g2.reshape(1, 32),
                  beta2.reshape(1, 32))
    return out[:n]


# speedup vs baseline: 41.5267x; 1.0627x over previous
import functools

import jax
import jax.numpy as jnp
from jax import lax
from jax.experimental import pallas as pl
from jax.experimental.pallas import tpu as pltpu
from jax.experimental.pallas import tpu_sc as plsc

BLK = 1024
NC = 2
NT = 16
U = 4


def _make_agg(two_tables, np_, n_rows):
    mesh = plsc.VectorSubcoreMesh(core_axis_name="c", subcore_axis_name="s")
    node_rows = np_ // NT
    stg = 256
    nfull, rem = divmod(node_rows, stg)
    if two_tables:
        tile_rows = n_rows // NT
    else:
        tile_rows = n_rows // (NC * NT)
    steps = tile_rows // U
    pairs = steps // 2

    def body(tab_hbm, sd_hbm, out_hbm, sd_v, rows_v, stage_v,
             acc_sh, sem_i, sem_g, sem_s):
        c = lax.axis_index("c")
        t = lax.axis_index("s")

        def zrow(i, carry):
            stage_v[i, :] = jnp.zeros((16,), jnp.float32)
            return carry
        lax.fori_loop(0, stg, zrow, 0)
        nbase = t * node_rows
        for k in range(nfull):
            pltpu.sync_copy(stage_v, acc_sh.at[pl.ds(nbase + k * stg, stg)])
        if rem:
            pltpu.sync_copy(stage_v.at[pl.ds(0, rem)],
                            acc_sh.at[pl.ds(nbase + nfull * stg, rem)])
        plsc.subcore_barrier()

        if two_tables:
            row0 = t * tile_rows
        else:
            row0 = (c * NT + t) * tile_rows
        off = c * np_

        def load_idx(g, buf):
            pltpu.async_copy(sd_hbm.at[pl.ds(row0 + g * U, U)],
                             sd_v.at[buf], sem_i)

        def idx_ready(buf):
            pltpu.make_async_copy(sd_hbm.at[pl.ds(row0, U)],
                                  sd_v.at[buf], sem_i).wait()

        def fire_gathers(buf):
            if two_tables:
                for j in range(U):
                    for k in range(8):
                        sl = (buf, j, 0, pl.ds(k * 16, 16))
                        sd_v[sl] = sd_v[sl] + off
            for j in range(U):
                pltpu.async_copy(tab_hbm.at[sd_v.at[buf, j, 0]],
                                 rows_v.at[buf, j], sem_g)

        def drain_gathers(buf):
            for j in range(U):
                pltpu.make_async_copy(tab_hbm.at[sd_v.at[buf, j, 0]],
                                      rows_v.at[buf, j], sem_g).wait()

        def scatter(buf):
            descs = [pltpu.async_copy(rows_v.at[buf, j],
                                      acc_sh.at[sd_v.at[buf, j, 1]], sem_s,
                                      add=True)
                     for j in range(U)]
            for dsc in descs:
                dsc.wait()

        load_idx(0, 0)
        idx_ready(0)
        fire_gathers(0)
        load_idx(1, 1)

        def pair(gg, carry):
            g0 = 2 * gg
            idx_ready(1)
            fire_gathers(1)
            drain_gathers(0)
            scatter(0)

            @pl.when(gg + 1 < pairs)
            def _():
                load_idx(g0 + 2, 0)

            drain_gathers(1)
            scatter(1)

            @pl.when(gg + 1 < pairs)
            def _():
                idx_ready(0)
                fire_gathers(0)
                load_idx(g0 + 3, 1)

            return carry
        lax.fori_loop(0, pairs, pair, 0)
        plsc.subcore_barrier()

        obase = c * np_ + nbase
        for k in range(nfull):
            pltpu.sync_copy(acc_sh.at[pl.ds(nbase + k * stg, stg)], stage_v)
            pltpu.sync_copy(stage_v, out_hbm.at[pl.ds(obase + k * stg, stg)])
        if rem:
            pltpu.sync_copy(acc_sh.at[pl.ds(nbase + nfull * stg, rem)],
                            stage_v.at[pl.ds(0, rem)])
            pltpu.sync_copy(stage_v.at[pl.ds(0, rem)],
                            out_hbm.at[pl.ds(obase + nfull * stg, rem)])

    return pl.kernel(
        body,
        out_type=jax.ShapeDtypeStruct((NC * np_, 16), jnp.float32),
        mesh=mesh,
        scratch_types=[
            pltpu.VMEM((2, U, 2, 128), jnp.int32),
            pltpu.VMEM((2, U, 128, 16), jnp.float32),
            pltpu.VMEM((256, 16), jnp.float32),
            pltpu.VMEM_SHARED((np_, 16), jnp.float32),
            pltpu.SemaphoreType.DMA,
            pltpu.SemaphoreType.DMA,
            pltpu.SemaphoreType.DMA,
        ],
        compiler_params=pltpu.CompilerParams(use_tc_tiling_on_sc=False),
    )


def _dense1_body(tab_ref, pa_ref, pb_ref, ws1t_ref, wn1t_ref, b1_ref, g1_ref,
                 bt1_ref, wn2t_ref, n_ref, h_ref, p_ref):
    ps = pa_ref[...] + pb_ref[...]
    cnt = ps[:, 12:13]
    inv = 1.0 / jnp.maximum(cnt, 1.0)
    agg = ps[:, :12] * inv
    xb = tab_ref[:, :12]
    h = (jnp.dot(xb, ws1t_ref[...], preferred_element_type=jnp.float32)
         + jnp.dot(agg, wn1t_ref[...], preferred_element_type=jnp.float32)
         + b1_ref[...])
    mu = jnp.mean(h, axis=-1, keepdims=True)
    var = jnp.mean((h - mu) ** 2, axis=-1, keepdims=True)
    h = (h - mu) * lax.rsqrt(var + 1e-5) * g1_ref[...] + bt1_ref[...]
    h = jnp.maximum(h, 0.0)
    row0 = pl.program_id(0) * BLK
    rows = row0 + lax.broadcasted_iota(jnp.int32, (BLK, 1), 0)
    h = jnp.where(rows < n_ref[0], h, 0.0)
    h_ref[...] = h
    p = jnp.dot(h, wn2t_ref[...], preferred_element_type=jnp.float32)
    p_ref[0] = p[:, :16]
    p_ref[1] = p[:, 16:]


def _dense2_body(h_ref, pa_ref, pb_ref, q0_ref, q1_ref, ws2t_ref, b2_ref,
                 g2_ref, bt2_ref, out_ref):
    cnt = pa_ref[:, 12:13] + pb_ref[:, 12:13]
    inv = 1.0 / jnp.maximum(cnt, 1.0)
    agg = jnp.concatenate([q0_ref[...], q1_ref[...]], axis=1) * inv
    h = (jnp.dot(h_ref[...], ws2t_ref[...], preferred_element_type=jnp.float32)
         + agg + b2_ref[...])
    mu = jnp.mean(h, axis=-1, keepdims=True)
    var = jnp.mean((h - mu) ** 2, axis=-1, keepdims=True)
    h = (h - mu) * lax.rsqrt(var + 1e-5) * g2_ref[...] + bt2_ref[...]
    out_ref[...] = jnp.maximum(h, 0.0)


def _blk_spec(cols, row_off=0):
    return pl.BlockSpec((BLK, cols), lambda i, _o=row_off: (i + _o, 0))


def _full_spec(shape):
    return pl.BlockSpec(shape, lambda i: tuple(0 for _ in shape))


def _dense1(np_, tab1, parts, ws1t, wn1t, b1, g1, bt1, wn2t, n_nodes):
    nb = np_ // BLK
    return pl.pallas_call(
        _dense1_body,
        grid=(nb,),
        in_specs=[
            _blk_spec(16), _blk_spec(16), _blk_spec(16, nb),
            _full_spec((12, 64)), _full_spec((12, 64)), _full_spec((1, 64)),
            _full_spec((1, 64)), _full_spec((1, 64)), _full_spec((64, 32)),
            pl.BlockSpec(memory_space=pltpu.SMEM),
        ],
        out_specs=[
            _blk_spec(64),
            pl.BlockSpec((2, BLK, 16), lambda i: (0, i, 0)),
        ],
        out_shape=[
            jax.ShapeDtypeStruct((np_, 64), jnp.float32),
            jax.ShapeDtypeStruct((2, np_, 16), jnp.float32),
        ],
    )(tab1, parts, parts, ws1t, wn1t, b1, g1, bt1, wn2t,
      jnp.full((1,), n_nodes, jnp.int32))


def _dense2(np_, h, parts, q, ws2t, b2, g2, bt2):
    nb = np_ // BLK
    return pl.pallas_call(
        _dense2_body,
        grid=(nb,),
        in_specs=[
            _blk_spec(64), _blk_spec(16), _blk_spec(16, nb),
            _blk_spec(16), _blk_spec(16, nb),
            _full_spec((64, 32)), _full_spec((1, 32)), _full_spec((1, 32)),
            _full_spec((1, 32)),
        ],
        out_specs=_blk_spec(32),
        out_shape=jax.ShapeDtypeStruct((np_, 32), jnp.float32),
    )(h, parts, parts, q, q, ws2t, b2, g2, bt2)


def kernel(x, edge_index, W_self1, W_neigh1, b1, g1, beta1,
           W_self2, W_neigh2, b2, g2, beta2):
    n = x.shape[0]
    np_ = -(-(n + 1) // BLK) * BLK
    s = edge_index[0]
    d = edge_index[1]
    e = s.shape[0]
    gran = NC * NT * 128 * U * 2
    e_pad = -(-e // gran) * gran
    r = e_pad // 128

    pad = jnp.full((e_pad - e,), n, jnp.int32)
    s2d = jnp.concatenate([s, pad]).reshape(r, 1, 128)
    d2d = jnp.concatenate([d, pad]).reshape(r, 1, 128)
    sd = jnp.concatenate([s2d, d2d], axis=1)

    tab1 = jnp.concatenate(
        [x, jnp.ones((n, 1), jnp.float32), jnp.zeros((n, 3), jnp.float32)],
        axis=1)
    tab1 = jnp.concatenate(
        [tab1, jnp.zeros((np_ - n, 16), jnp.float32)], axis=0)

    parts = _make_agg(False, np_, r)(tab1, sd)

    h, p = _dense1(np_, tab1, parts,
                   W_self1.T, W_neigh1.T, b1.reshape(1, 64),
                   g1.reshape(1, 64), beta1.reshape(1, 64),
                   W_neigh2.T, n)

    tab2 = p.reshape(2 * np_, 16)
    q = _make_agg(True, np_, r)(tab2, sd)

    out = _dense2(np_, h, parts, q,
                  W_self2.T, b2.reshape(1, 32), g2.reshape(1, 32),
                  beta2.reshape(1, 32))
    return out[:n]


# speedup vs baseline: 46.0847x; 1.1098x over previous
import functools

import jax
import jax.numpy as jnp
from jax import lax
from jax.experimental import pallas as pl
from jax.experimental.pallas import tpu as pltpu
from jax.experimental.pallas import tpu_sc as plsc

BLK = 1024
NC = 2
NT = 16
U = 4


def _make_agg(two_tables, np_, n_rows):
    mesh = plsc.VectorSubcoreMesh(core_axis_name="c", subcore_axis_name="s")
    node_rows = np_ // NT
    stg = 256
    nfull, rem = divmod(node_rows, stg)
    if two_tables:
        tile_rows = n_rows // NT
    else:
        tile_rows = n_rows // (NC * NT)
    steps = tile_rows // U
    pairs = steps // 2

    def body(tab_hbm, sd_hbm, out_hbm, sd_v, rows_v, stage_v,
             acc_sh, sem_i, sem_g, sem_s):
        c = lax.axis_index("c")
        t = lax.axis_index("s")

        def zrow(i, carry):
            stage_v[i, :] = jnp.zeros((16,), jnp.float32)
            return carry
        lax.fori_loop(0, stg, zrow, 0)
        nbase = t * node_rows
        for k in range(nfull):
            pltpu.sync_copy(stage_v, acc_sh.at[pl.ds(nbase + k * stg, stg)])
        if rem:
            pltpu.sync_copy(stage_v.at[pl.ds(0, rem)],
                            acc_sh.at[pl.ds(nbase + nfull * stg, rem)])
        plsc.subcore_barrier()

        if two_tables:
            row0 = t * tile_rows
        else:
            row0 = (c * NT + t) * tile_rows
        off = c * np_

        def load_idx(g, buf):
            pltpu.async_copy(sd_hbm.at[pl.ds(row0 + g * U, U)],
                             sd_v.at[buf], sem_i)

        def idx_ready(buf):
            pltpu.make_async_copy(sd_hbm.at[pl.ds(row0, U)],
                                  sd_v.at[buf], sem_i).wait()

        def fire_gathers(buf):
            if two_tables:
                for j in range(U):
                    for k in range(8):
                        sl = (buf, j, 0, pl.ds(k * 16, 16))
                        sd_v[sl] = sd_v[sl] + off
            for j in range(U):
                pltpu.async_copy(tab_hbm.at[sd_v.at[buf, j, 0]],
                                 rows_v.at[buf, j], sem_g)

        def drain_gathers(buf):
            for j in range(U):
                pltpu.make_async_copy(tab_hbm.at[sd_v.at[buf, j, 0]],
                                      rows_v.at[buf, j], sem_g).wait()

        def scatter(buf):
            descs = [pltpu.async_copy(rows_v.at[buf, j],
                                      acc_sh.at[sd_v.at[buf, j, 1]], sem_s,
                                      add=True)
                     for j in range(U)]
            for dsc in descs:
                dsc.wait()

        load_idx(0, 0)
        idx_ready(0)
        fire_gathers(0)
        load_idx(1, 1)

        def pair(gg, carry):
            g0 = 2 * gg
            idx_ready(1)
            fire_gathers(1)
            drain_gathers(0)
            scatter(0)

            @pl.when(gg + 1 < pairs)
            def _():
                load_idx(g0 + 2, 0)

            drain_gathers(1)
            scatter(1)

            @pl.when(gg + 1 < pairs)
            def _():
                idx_ready(0)
                fire_gathers(0)
                load_idx(g0 + 3, 1)

            return carry
        lax.fori_loop(0, pairs, pair, 0)
        plsc.subcore_barrier()

        obase = c * np_ + nbase
        for k in range(nfull):
            pltpu.sync_copy(acc_sh.at[pl.ds(nbase + k * stg, stg)], stage_v)
            pltpu.sync_copy(stage_v, out_hbm.at[pl.ds(obase + k * stg, stg)])
        if rem:
            pltpu.sync_copy(acc_sh.at[pl.ds(nbase + nfull * stg, rem)],
                            stage_v.at[pl.ds(0, rem)])
            pltpu.sync_copy(stage_v.at[pl.ds(0, rem)],
                            out_hbm.at[pl.ds(obase + nfull * stg, rem)])

    return pl.kernel(
        body,
        out_type=jax.ShapeDtypeStruct((NC * np_, 16), jnp.float32),
        mesh=mesh,
        scratch_types=[
            pltpu.VMEM((2, U, 2, 128), jnp.int32),
            pltpu.VMEM((2, U, 128, 16), jnp.float32),
            pltpu.VMEM((256, 16), jnp.float32),
            pltpu.VMEM_SHARED((np_, 16), jnp.float32),
            pltpu.SemaphoreType.DMA,
            pltpu.SemaphoreType.DMA,
            pltpu.SemaphoreType.DMA,
        ],
        compiler_params=pltpu.CompilerParams(use_tc_tiling_on_sc=False),
    )


def _dot(a, b):
    return jnp.dot(a, b, preferred_element_type=jnp.float32)


def _dense1_body(tab_ref, pa_ref, pb_ref, wsb_ref, wnb_ref, cc_ref, m64_ref,
                 o64_ref, b1_ref, g1_ref, bt1_ref, wn2b_ref, sel0_ref,
                 sel1_ref, n_ref, h_ref, p_ref):
    ps = pa_ref[...] + pb_ref[...]
    cnt8 = _dot(ps, cc_ref[...])
    inv8 = 1.0 / jnp.maximum(cnt8, 1.0)
    h = (_dot(tab_ref[...], wsb_ref[...])
         + _dot(ps, wnb_ref[...]) * _dot(inv8, m64_ref[...])
         + b1_ref[...])
    mean8 = _dot(h, o64_ref[...])
    var8 = _dot(h * h, o64_ref[...]) - mean8 * mean8
    h = ((h - _dot(mean8, m64_ref[...]))
         * _dot(lax.rsqrt(var8 + 1e-5), m64_ref[...]) * g1_ref[...]
         + bt1_ref[...])
    h = jnp.maximum(h, 0.0)
    nid = (pl.program_id(0) * BLK
           + 8 * lax.broadcasted_iota(jnp.int32, (BLK // 8, 8), 0)
           + lax.broadcasted_iota(jnp.int32, (BLK // 8, 8), 1))
    mask8 = jnp.where(nid < n_ref[0], 1.0, 0.0)
    h = h * _dot(mask8, m64_ref[...])
    h_ref[...] = h
    p8 = _dot(h, wn2b_ref[...])
    p_ref[0] = _dot(p8, sel0_ref[...])
    p_ref[1] = _dot(p8, sel1_ref[...])


def _dense2_body(h_ref, pa_ref, pb_ref, q0_ref, q1_ref, ws2b_ref, cc_ref,
                 a0_ref, a1_ref, m32_ref, o32_ref, b2_ref, g2_ref, bt2_ref,
                 out_ref):
    ps = pa_ref[...] + pb_ref[...]
    cnt8 = _dot(ps, cc_ref[...])
    inv8 = 1.0 / jnp.maximum(cnt8, 1.0)
    agg = ((_dot(q0_ref[...], a0_ref[...]) + _dot(q1_ref[...], a1_ref[...]))
           * _dot(inv8, m32_ref[...]))
    h = _dot(h_ref[...], ws2b_ref[...]) + agg + b2_ref[...]
    mean8 = _dot(h, o32_ref[...])
    var8 = _dot(h * h, o32_ref[...]) - mean8 * mean8
    h = ((h - _dot(mean8, m32_ref[...]))
         * _dot(lax.rsqrt(var8 + 1e-5), m32_ref[...]) * g2_ref[...]
         + bt2_ref[...])
    out_ref[...] = jnp.maximum(h, 0.0)


def _blk_spec(cols, row_off=0):
    return pl.BlockSpec((BLK, cols), lambda i, _o=row_off: (i + _o, 0))


def _pk_spec(blk_off=0):
    return pl.BlockSpec((BLK // 8, 128), lambda i, _o=blk_off: (i + _o, 0))


def _full_spec(shape):
    return pl.BlockSpec(shape, lambda i: tuple(0 for _ in shape))


def _dense1(np_, tab1_pk, parts_pk, consts, n_nodes):
    nb = np_ // BLK
    return pl.pallas_call(
        _dense1_body,
        grid=(nb,),
        in_specs=[
            _pk_spec(), _pk_spec(), _pk_spec(nb),
        ] + [_full_spec(c.shape) for c in consts] + [
            pl.BlockSpec(memory_space=pltpu.SMEM),
        ],
        out_specs=[
            pl.BlockSpec((BLK // 8, 512), lambda i: (i, 0)),
            pl.BlockSpec((2, BLK // 8, 128), lambda i: (0, i, 0)),
        ],
        out_shape=[
            jax.ShapeDtypeStruct((np_ // 8, 512), jnp.float32),
            jax.ShapeDtypeStruct((2, np_ // 8, 128), jnp.float32),
        ],
    )(tab1_pk, parts_pk, parts_pk, *consts,
      jnp.full((1,), n_nodes, jnp.int32))


def _dense2(np_, n, h_pk, parts_pk, q_pk, consts):
    nb = np_ // BLK
    return pl.pallas_call(
        _dense2_body,
        grid=(nb,),
        in_specs=[
            pl.BlockSpec((BLK // 8, 512), lambda i: (i, 0)),
            _pk_spec(), _pk_spec(nb), _pk_spec(), _pk_spec(nb),
        ] + [_full_spec(c.shape) for c in consts],
        out_specs=pl.BlockSpec((BLK // 8, 256), lambda i: (i, 0)),
        out_shape=jax.ShapeDtypeStruct((n // 8, 256), jnp.float32),
    )(h_pk, parts_pk, parts_pk, q_pk, q_pk, *consts)


def kernel(x, edge_index, W_self1, W_neigh1, b1, g1, beta1,
           W_self2, W_neigh2, b2, g2, beta2):
    n = x.shape[0]
    np_ = -(-(n + 1) // BLK) * BLK
    s = edge_index[0]
    d = edge_index[1]
    e = s.shape[0]
    gran = NC * NT * 128 * U * 2
    e_pad = -(-e // gran) * gran
    r = e_pad // 128

    pad = jnp.full((e_pad - e,), n, jnp.int32)
    s2d = jnp.concatenate([s, pad]).reshape(r, 1, 128)
    d2d = jnp.concatenate([d, pad]).reshape(r, 1, 128)
    sd = jnp.concatenate([s2d, d2d], axis=1)

    tab1 = jnp.concatenate(
        [x, jnp.ones((n, 1), jnp.float32), jnp.zeros((n, 3), jnp.float32)],
        axis=1)
    tab1 = jnp.concatenate(
        [tab1, jnp.zeros((np_ - n, 16), jnp.float32)], axis=0)

    parts = _make_agg(False, np_, r)(tab1, sd)

    eye8 = jnp.eye(8, dtype=jnp.float32)
    ws_blk = jnp.kron(eye8, jnp.pad(W_self1.T, ((0, 4), (0, 0))))
    wn_blk = jnp.kron(eye8, jnp.pad(W_neigh1.T, ((0, 4), (0, 0))))
    cc = jnp.kron(eye8, jnp.zeros((16, 1), jnp.float32).at[12, 0].set(1.0))
    m64 = jnp.kron(eye8, jnp.ones((1, 64), jnp.float32))
    o64 = jnp.kron(eye8, jnp.full((64, 1), 1.0 / 64, jnp.float32))
    wn2_blk = jnp.kron(eye8, W_neigh2.T)
    i16 = jnp.eye(16, dtype=jnp.float32)
    sel0 = jnp.kron(eye8, jnp.concatenate(
        [i16, jnp.zeros((16, 16), jnp.float32)], axis=0))
    sel1 = jnp.kron(eye8, jnp.concatenate(
        [jnp.zeros((16, 16), jnp.float32), i16], axis=0))
    c1 = [ws_blk, wn_blk, cc, m64, o64,
          jnp.tile(b1, 8).reshape(1, 512), jnp.tile(g1, 8).reshape(1, 512),
          jnp.tile(beta1, 8).reshape(1, 512), wn2_blk, sel0, sel1]

    h_pk, p = _dense1(np_, tab1.reshape(np_ // 8, 128),
                      parts.reshape(2 * np_ // 8, 128), c1, n)

    tab2 = p.reshape(2 * np_, 16)
    q = _make_agg(True, np_, r)(tab2, sd)

    ws2_blk = jnp.kron(eye8, W_self2.T)
    a0 = jnp.kron(eye8, jnp.concatenate(
        [i16, jnp.zeros((16, 16), jnp.float32)], axis=1))
    a1 = jnp.kron(eye8, jnp.concatenate(
        [jnp.zeros((16, 16), jnp.float32), i16], axis=1))
    m32 = jnp.kron(eye8, jnp.ones((1, 32), jnp.float32))
    o32 = jnp.kron(eye8, jnp.full((32, 1), 1.0 / 32, jnp.float32))
    c2 = [ws2_blk, cc, a0, a1, m32, o32,
          jnp.tile(b2, 8).reshape(1, 256), jnp.tile(g2, 8).reshape(1, 256),
          jnp.tile(beta2, 8).reshape(1, 256)]

    out_pk = _dense2(np_, n, h_pk, parts.reshape(2 * np_ // 8, 128),
                     q.reshape(2 * np_ // 8, 128), c2)
    return out_pk.reshape(n, 32)


# speedup vs baseline: 48.7555x; 1.0580x over previous
import functools

import jax
import jax.numpy as jnp
from jax import lax
from jax.experimental import pallas as pl
from jax.experimental.pallas import tpu as pltpu
from jax.experimental.pallas import tpu_sc as plsc

BLK = 1024
NC = 2
NT = 16
U = 4


def _make_agg(two_tables, np_, n_rows):
    mesh = plsc.VectorSubcoreMesh(core_axis_name="c", subcore_axis_name="s")
    node_rows = np_ // NT
    stg = 112
    nfull, rem = divmod(node_rows, stg)
    if two_tables:
        tile_rows = n_rows // NT
    else:
        tile_rows = n_rows // (NC * NT)
    steps = tile_rows // U

    def body(tab_hbm, sd_hbm, out_hbm, sd_v, rows_v, stage_v, acc_sh,
             sem_i0, sem_i1, sem_i2, sem_g0, sem_g1, sem_g2,
             sem_s0, sem_s1, sem_s2):
        sem_i = [sem_i0, sem_i1, sem_i2]
        sem_g = [sem_g0, sem_g1, sem_g2]
        sem_s = [sem_s0, sem_s1, sem_s2]
        c = lax.axis_index("c")
        t = lax.axis_index("s")

        def zrow(i, carry):
            stage_v[i, :] = jnp.zeros((16,), jnp.float32)
            return carry
        lax.fori_loop(0, stg, zrow, 0)
        nbase = t * node_rows
        for k in range(nfull):
            pltpu.sync_copy(stage_v, acc_sh.at[pl.ds(nbase + k * stg, stg)])
        if rem:
            pltpu.sync_copy(stage_v.at[pl.ds(0, rem)],
                            acc_sh.at[pl.ds(nbase + nfull * stg, rem)])
        plsc.subcore_barrier()

        if two_tables:
            row0 = t * tile_rows
        else:
            row0 = (c * NT + t) * tile_rows
        off = c * np_

        def load_idx(g, b):
            pltpu.async_copy(sd_hbm.at[pl.ds(row0 + g * U, U)],
                             sd_v.at[b], sem_i[b])

        def idx_ready(b):
            pltpu.make_async_copy(sd_hbm.at[pl.ds(row0, U)],
                                  sd_v.at[b], sem_i[b]).wait()

        def fire_gathers(b):
            if two_tables:
                for j in range(U):
                    for k in range(8):
                        sl = (b, j, 0, pl.ds(k * 16, 16))
                        sd_v[sl] = sd_v[sl] + off
            for j in range(U):
                pltpu.async_copy(tab_hbm.at[sd_v.at[b, j, 0]],
                                 rows_v.at[b, j], sem_g[b])

        def drain_gathers(b):
            for j in range(U):
                pltpu.make_async_copy(tab_hbm.at[sd_v.at[b, j, 0]],
                                      rows_v.at[b, j], sem_g[b]).wait()

        def scatter_fire(b):
            for j in range(U):
                pltpu.async_copy(rows_v.at[b, j],
                                 acc_sh.at[sd_v.at[b, j, 1]], sem_s[b],
                                 add=True)

        def scatter_wait(b):
            for j in range(U):
                pltpu.make_async_copy(rows_v.at[b, j],
                                      acc_sh.at[sd_v.at[b, j, 1]],
                                      sem_s[b]).wait()

        load_idx(0, 0)
        load_idx(1, 1)
        idx_ready(0)
        fire_gathers(0)

        def tri(m, carry):
            g0 = 3 * m
            for k in range(3):
                b, b1, b2 = k % 3, (k + 1) % 3, (k + 2) % 3
                g = g0 + k

                @pl.when(g + 1 < steps)
                def _():
                    idx_ready(b1)
                    fire_gathers(b1)

                drain_gathers(b)
                scatter_fire(b)

                @pl.when(g > 0)
                def _():
                    scatter_wait(b2)

                @pl.when(g + 2 < steps)
                def _():
                    load_idx(g + 2, b2)

            return carry
        lax.fori_loop(0, steps // 3, tri, 0)
        scatter_wait((steps - 1) % 3)
        plsc.subcore_barrier()

        obase = c * np_ + nbase
        for k in range(nfull):
            pltpu.sync_copy(acc_sh.at[pl.ds(nbase + k * stg, stg)], stage_v)
            pltpu.sync_copy(stage_v, out_hbm.at[pl.ds(obase + k * stg, stg)])
        if rem:
            pltpu.sync_copy(acc_sh.at[pl.ds(nbase + nfull * stg, rem)],
                            stage_v.at[pl.ds(0, rem)])
            pltpu.sync_copy(stage_v.at[pl.ds(0, rem)],
                            out_hbm.at[pl.ds(obase + nfull * stg, rem)])

    return pl.kernel(
        body,
        out_type=jax.ShapeDtypeStruct((NC * np_, 16), jnp.float32),
        mesh=mesh,
        scratch_types=[
            pltpu.VMEM((3, U, 2, 128), jnp.int32),
            pltpu.VMEM((3, U, 128, 16), jnp.float32),
            pltpu.VMEM((112, 16), jnp.float32),
            pltpu.VMEM_SHARED((np_, 16), jnp.float32),
        ] + [pltpu.SemaphoreType.DMA] * 9,
        compiler_params=pltpu.CompilerParams(use_tc_tiling_on_sc=False),
    )


def _dot(a, b):
    return jnp.dot(a, b, preferred_element_type=jnp.float32)


def _dense1_body(tab_ref, pa_ref, pb_ref, wsb_ref, wnb_ref, cc_ref, m64_ref,
                 o64_ref, b1_ref, g1_ref, bt1_ref, wn2b_ref, sel0_ref,
                 sel1_ref, n_ref, h_ref, p_ref):
    ps = pa_ref[...] + pb_ref[...]
    cnt8 = _dot(ps, cc_ref[...])
    inv8 = 1.0 / jnp.maximum(cnt8, 1.0)
    h = (_dot(tab_ref[...], wsb_ref[...])
         + _dot(ps, wnb_ref[...]) * _dot(inv8, m64_ref[...])
         + b1_ref[...])
    mean8 = _dot(h, o64_ref[...])
    var8 = _dot(h * h, o64_ref[...]) - mean8 * mean8
    h = ((h - _dot(mean8, m64_ref[...]))
         * _dot(lax.rsqrt(var8 + 1e-5), m64_ref[...]) * g1_ref[...]
         + bt1_ref[...])
    h = jnp.maximum(h, 0.0)
    nid = (pl.program_id(0) * BLK
           + 8 * lax.broadcasted_iota(jnp.int32, (BLK // 8, 8), 0)
           + lax.broadcasted_iota(jnp.int32, (BLK // 8, 8), 1))
    mask8 = jnp.where(nid < n_ref[0], 1.0, 0.0)
    h = h * _dot(mask8, m64_ref[...])
    h_ref[...] = h
    p8 = _dot(h, wn2b_ref[...])
    p_ref[0] = _dot(p8, sel0_ref[...])
    p_ref[1] = _dot(p8, sel1_ref[...])


def _dense2_body(h_ref, pa_ref, pb_ref, q0_ref, q1_ref, ws2b_ref, cc_ref,
                 a0_ref, a1_ref, m32_ref, o32_ref, b2_ref, g2_ref, bt2_ref,
                 out_ref):
    ps = pa_ref[...] + pb_ref[...]
    cnt8 = _dot(ps, cc_ref[...])
    inv8 = 1.0 / jnp.maximum(cnt8, 1.0)
    agg = ((_dot(q0_ref[...], a0_ref[...]) + _dot(q1_ref[...], a1_ref[...]))
           * _dot(inv8, m32_ref[...]))
    h = _dot(h_ref[...], ws2b_ref[...]) + agg + b2_ref[...]
    mean8 = _dot(h, o32_ref[...])
    var8 = _dot(h * h, o32_ref[...]) - mean8 * mean8
    h = ((h - _dot(mean8, m32_ref[...]))
         * _dot(lax.rsqrt(var8 + 1e-5), m32_ref[...]) * g2_ref[...]
         + bt2_ref[...])
    out_ref[...] = jnp.maximum(h, 0.0)


def _blk_spec(cols, row_off=0):
    return pl.BlockSpec((BLK, cols), lambda i, _o=row_off: (i + _o, 0))


def _pk_spec(blk_off=0):
    return pl.BlockSpec((BLK // 8, 128), lambda i, _o=blk_off: (i + _o, 0))


def _full_spec(shape):
    return pl.BlockSpec(shape, lambda i: tuple(0 for _ in shape))


def _dense1(np_, tab1_pk, parts_pk, consts, n_nodes):
    nb = np_ // BLK
    return pl.pallas_call(
        _dense1_body,
        grid=(nb,),
        in_specs=[
            _pk_spec(), _pk_spec(), _pk_spec(nb),
        ] + [_full_spec(c.shape) for c in consts] + [
            pl.BlockSpec(memory_space=pltpu.SMEM),
        ],
        out_specs=[
            pl.BlockSpec((BLK // 8, 512), lambda i: (i, 0)),
            pl.BlockSpec((2, BLK // 8, 128), lambda i: (0, i, 0)),
        ],
        out_shape=[
            jax.ShapeDtypeStruct((np_ // 8, 512), jnp.float32),
            jax.ShapeDtypeStruct((2, np_ // 8, 128), jnp.float32),
        ],
    )(tab1_pk, parts_pk, parts_pk, *consts,
      jnp.full((1,), n_nodes, jnp.int32))


def _dense2(np_, n, h_pk, parts_pk, q_pk, consts):
    nb = np_ // BLK
    return pl.pallas_call(
        _dense2_body,
        grid=(nb,),
        in_specs=[
            pl.BlockSpec((BLK // 8, 512), lambda i: (i, 0)),
            _pk_spec(), _pk_spec(nb), _pk_spec(), _pk_spec(nb),
        ] + [_full_spec(c.shape) for c in consts],
        out_specs=pl.BlockSpec((BLK // 8, 256), lambda i: (i, 0)),
        out_shape=jax.ShapeDtypeStruct((n // 8, 256), jnp.float32),
    )(h_pk, parts_pk, parts_pk, q_pk, q_pk, *consts)


def kernel(x, edge_index, W_self1, W_neigh1, b1, g1, beta1,
           W_self2, W_neigh2, b2, g2, beta2):
    n = x.shape[0]
    np_ = -(-(n + 1) // BLK) * BLK
    s = edge_index[0]
    d = edge_index[1]
    e = s.shape[0]
    gran = NC * NT * 128 * U * 3
    e_pad = -(-e // gran) * gran
    r = e_pad // 128

    pad = jnp.full((e_pad - e,), n, jnp.int32)
    s2d = jnp.concatenate([s, pad]).reshape(r, 1, 128)
    d2d = jnp.concatenate([d, pad]).reshape(r, 1, 128)
    sd = jnp.concatenate([s2d, d2d], axis=1)

    tab1 = jnp.concatenate(
        [x, jnp.ones((n, 1), jnp.float32), jnp.zeros((n, 3), jnp.float32)],
        axis=1)
    tab1 = jnp.concatenate(
        [tab1, jnp.zeros((np_ - n, 16), jnp.float32)], axis=0)

    parts = _make_agg(False, np_, r)(tab1, sd)

    eye8 = jnp.eye(8, dtype=jnp.float32)
    ws_blk = jnp.kron(eye8, jnp.pad(W_self1.T, ((0, 4), (0, 0))))
    wn_blk = jnp.kron(eye8, jnp.pad(W_neigh1.T, ((0, 4), (0, 0))))
    cc = jnp.kron(eye8, jnp.zeros((16, 1), jnp.float32).at[12, 0].set(1.0))
    m64 = jnp.kron(eye8, jnp.ones((1, 64), jnp.float32))
    o64 = jnp.kron(eye8, jnp.full((64, 1), 1.0 / 64, jnp.float32))
    wn2_blk = jnp.kron(eye8, W_neigh2.T)
    i16 = jnp.eye(16, dtype=jnp.float32)
    sel0 = jnp.kron(eye8, jnp.concatenate(
        [i16, jnp.zeros((16, 16), jnp.float32)], axis=0))
    sel1 = jnp.kron(eye8, jnp.concatenate(
        [jnp.zeros((16, 16), jnp.float32), i16], axis=0))
    c1 = [ws_blk, wn_blk, cc, m64, o64,
          jnp.tile(b1, 8).reshape(1, 512), jnp.tile(g1, 8).reshape(1, 512),
          jnp.tile(beta1, 8).reshape(1, 512), wn2_blk, sel0, sel1]

    h_pk, p = _dense1(np_, tab1.reshape(np_ // 8, 128),
                      parts.reshape(2 * np_ // 8, 128), c1, n)

    tab2 = p.reshape(2 * np_, 16)
    q = _make_agg(True, np_, r)(tab2, sd)

    ws2_blk = jnp.kron(eye8, W_self2.T)
    a0 = jnp.kron(eye8, jnp.concatenate(
        [i16, jnp.zeros((16, 16), jnp.float32)], axis=1))
    a1 = jnp.kron(eye8, jnp.concatenate(
        [jnp.zeros((16, 16), jnp.float32), i16], axis=1))
    m32 = jnp.kron(eye8, jnp.ones((1, 32), jnp.float32))
    o32 = jnp.kron(eye8, jnp.full((32, 1), 1.0 / 32, jnp.float32))
    c2 = [ws2_blk, cc, a0, a1, m32, o32,
          jnp.tile(b2, 8).reshape(1, 256), jnp.tile(g2, 8).reshape(1, 256),
          jnp.tile(beta2, 8).reshape(1, 256)]

    out_pk = _dense2(np_, n, h_pk, parts.reshape(2 * np_ // 8, 128),
                     q.reshape(2 * np_ // 8, 128), c2)
    return out_pk.reshape(n, 32)
